# baseline (speedup 1.0000x reference)
import functools

import jax
import jax.numpy as jnp
from jax import lax
from jax.experimental import pallas as pl
from jax.experimental.pallas import tpu as pltpu
from jax.experimental.pallas import tpu_sc as plsc

D_MODEL = 1024
BATCH = 4
SEQ = 8192

NC = 2
NS = 16
NW = NC * NS
ROWS = BATCH * SEQ
ROWS_PER_W = ROWS // NW
W_PER_BATCH = SEQ // ROWS_PER_W
R = 16
T = ROWS_PER_W // R
L = 16
VPB = ROWS_PER_W // L

_mesh = plsc.VectorSubcoreMesh(core_axis_name="c", subcore_axis_name="s")


@functools.partial(
    pl.kernel,
    mesh=_mesh,
    out_type=jax.ShapeDtypeStruct((ROWS, D_MODEL), jnp.float32),
    scratch_types=[
        pltpu.VMEM((SEQ,), jnp.int32),
        pltpu.VMEM((T, R), jnp.int32),
        pltpu.VMEM((R, D_MODEL), jnp.float32),
        pltpu.VMEM((R, D_MODEL), jnp.float32),
        pltpu.SemaphoreType.DMA,
    ],
    compiler_params=pltpu.CompilerParams(needs_layout_passes=False),
)
def _pos_emb_kernel(x_hbm, mask_hbm, table_hbm, out_hbm,
                    maskrow, idx, xbuf, tbuf, sem):
    wid = lax.axis_index("s") * NC + lax.axis_index("c")
    batch = wid // W_PER_BATCH
    sub = wid % W_PER_BATCH
    base = wid * ROWS_PER_W

    pltpu.sync_copy(mask_hbm.at[batch], maskrow)

    def pre_body(i, acc):
        return acc + maskrow[pl.ds(i * L, L)]
    acc = lax.fori_loop(0, sub * VPB, pre_body,
                        jnp.zeros((L,), jnp.int32))
    carry0 = jnp.sum(acc)

    def ids_body(j, carry):
        v = maskrow[pl.ds((sub * VPB + j) * L, L)]
        cs = plsc.cumsum(v)
        idx[j, :] = (cs + carry) * v
        return carry + jnp.sum(v)
    lax.fori_loop(0, VPB, ids_body, carry0)

    def chunk_body(t, _):
        row0 = base + t * R
        pltpu.sync_copy(x_hbm.at[pl.ds(row0, R)], xbuf)
        pltpu.async_copy(table_hbm.at[idx.at[t]], tbuf, sem).wait()

        def add_row(r, _2):
            for c in range(D_MODEL // L):
                sl = pl.ds(c * L, L)
                plsc.addupdate(xbuf.at[r, sl], tbuf[r, sl])
            return 0
        lax.fori_loop(0, R, add_row, 0)
        pltpu.sync_copy(xbuf, out_hbm.at[pl.ds(row0, R)])
        return 0
    lax.fori_loop(0, T, chunk_body, 0)


def kernel(x, mask, pos_table):
    x2 = x.reshape(ROWS, D_MODEL)
    out = _pos_emb_kernel(x2, mask, pos_table)
    return out.reshape(BATCH, SEQ, D_MODEL)
